# Initial kernel scaffold; baseline (speedup 1.0000x reference)
import jax
import jax.numpy as jnp
from jax.experimental import pallas as pl


def kernel(sensor_input, initial_pose, initial_angles, wheel_ticks, conv_w, conv_b, lin_w, lin_b):
    raise NotImplementedError("write your pallas kernel here")


# speedup vs baseline: 3.3477x; 3.3477x over previous
import jax
import jax.numpy as jnp
from jax.experimental import pallas as pl

_NUM_IK_STEPS = 9


def _fused_body(xr_ref, w_ref, b_ref, lwa_ref, lwb_ref, s_ref,
                traj_ref, pose_ref):
    cf = jnp.dot(w_ref[...], xr_ref[...], preferred_element_type=jnp.float32)
    cf = jnp.maximum(cf + b_ref[...], 0.0)
    pool = jnp.maximum(jnp.maximum(cf[:, 0:64], cf[:, 64:128]),
                       jnp.maximum(cf[:, 128:192], cf[:, 192:256]))
    t0 = jnp.sum(pool * lwa_ref[...], keepdims=True) + s_ref[:, 7:8]
    t1 = jnp.sum(pool * lwb_ref[...], keepdims=True) + s_ref[:, 8:9]

    q1 = s_ref[:, 3:4]
    q2 = s_ref[:, 4:5]
    for _ in range(_NUM_IK_STEPS):
        s1, c1 = jnp.sin(q1), jnp.cos(q1)
        q12 = q1 + q2
        s12, c12 = jnp.sin(q12), jnp.cos(q12)
        ex = t0 - (c1 + c12)
        ey = t1 - (s1 + s12)
        j11 = -s1 - s12
        j12 = -s12
        j21 = c1 + c12
        j22 = c12
        inv = 1.0 / (j11 * j22 - j12 * j21 + 1e-6)
        q1 = q1 + (j22 * ex - j12 * ey) * inv
        q2 = q2 + (j11 * ey - j21 * ex) * inv

    s1, c1 = jnp.sin(q1), jnp.cos(q1)
    q12 = q1 + q2
    s12, c12 = jnp.sin(q12), jnp.cos(q12)
    sx = c1 + c12
    sy = s1 + s12
    kf = jax.lax.broadcasted_iota(jnp.int32, (2, 32), 1).astype(jnp.float32)
    row = jax.lax.broadcasted_iota(jnp.int32, (2, 32), 0)
    t1v = jnp.minimum(kf, 9.0) * (1.0 / 9.0)
    t2v = jnp.maximum(kf - 9.0, 0.0) * (1.0 / 9.0)
    base = jnp.where(row == 0, sx, sy)
    traj_ref[...] = (base + jnp.where(row == 0, -0.2, 0.0) * t1v
                     + jnp.where(row == 0, 0.0, -0.1) * t2v)

    d_l = s_ref[:, 5:6] * 1e-4
    d_r = s_ref[:, 6:7] * 1e-4
    d_c = (d_l + d_r) * 0.5
    d_th = (d_r - d_l) * 2.0
    avg = s_ref[:, 2:3] + d_th * 0.5
    npx = s_ref[:, 0:1] + d_c * jnp.cos(avg)
    npy = s_ref[:, 1:2] + d_c * jnp.sin(avg)
    npth = s_ref[:, 2:3] + d_th
    lane = jax.lax.broadcasted_iota(jnp.int32, (1, 8), 1)
    pose_ref[...] = jnp.where(lane == 0, npx,
                              jnp.where(lane == 1, npy, npth))


def kernel(sensor_input, initial_pose, initial_angles, wheel_ticks,
           conv_w, conv_b, lin_w, lin_b):
    x = sensor_input[0]
    xpad = jnp.pad(x, ((0, 0), (1, 0), (1, 0)))[:, :64, :64]
    xr = xpad.reshape(3, 16, 4, 16, 4).transpose(0, 2, 4, 1, 3)
    xr = xr.reshape(48, 16, 16)
    xr = xr.reshape(48, 8, 2, 8, 2).transpose(0, 2, 4, 1, 3).reshape(48, 256)
    w_mat = conv_w.reshape(4, 48)
    b_col = conv_b.reshape(4, 1)
    lw = lin_w.reshape(2, 4, 64)
    scalars = jnp.concatenate([
        initial_pose, initial_angles, wheel_ticks, lin_b,
        jnp.zeros((7,), jnp.float32)]).reshape(1, 16)

    traj_raw, pose_raw = pl.pallas_call(
        _fused_body,
        out_shape=[jax.ShapeDtypeStruct((2, 32), jnp.float32),
                   jax.ShapeDtypeStruct((1, 8), jnp.float32)],
    )(xr, w_mat, b_col, lw[0], lw[1], scalars)

    trajectory = traj_raw[:, :19].T
    new_pose = pose_raw[0, :3]
    return trajectory, new_pose


# speedup vs baseline: 4.8737x; 1.4559x over previous
import jax
import jax.numpy as jnp
from jax.experimental import pallas as pl
from jax.experimental.pallas import tpu as pltpu

_NUM_IK_STEPS = 9


def _conv_pool_body(x_ref, cw_ref, cb_ref, pool_ref):
    ri = jax.lax.broadcasted_iota(jnp.int32, (64, 72), 0)
    mi = jax.lax.broadcasted_iota(jnp.int32, (64, 72), 1)
    tgt = jnp.where(mi < 64, 8 * (mi % 8) + mi // 8, 8 * (mi - 64) - 1)
    sel = (ri == tgt).astype(jnp.float32)

    def colgroup(y, coff):
        if coff >= 0:
            return y[:, coff * 8:coff * 8 + 8]
        return y[:, 64:72]

    acc = [[None] * 4 for _ in range(4)]
    zrow = jnp.zeros((1, 64), jnp.float32)
    for ci in range(3):
        rows = [x_ref[pl.ds(ci * 64 + r, 8, 8), :]
                .astype(jnp.bfloat16).astype(jnp.float32) for r in range(8)]
        rows_m1 = jnp.concatenate([zrow, rows[7][:7, :]], axis=0)
        ys = {-1: jnp.dot(rows_m1, sel, preferred_element_type=jnp.float32,
                          precision=jax.lax.Precision.HIGHEST)}
        for r in range(7):
            ys[r] = jnp.dot(rows[r], sel, preferred_element_type=jnp.float32,
                            precision=jax.lax.Precision.HIGHEST)
        for di in range(2):
            for dj in range(2):
                g = di * 2 + dj
                for dy in range(4):
                    for dx in range(4):
                        t = colgroup(ys[4 * di + dy - 1], 4 * dj + dx - 1)
                        for co in range(4):
                            w = cw_ref[co, ci, dy, dx].astype(
                                jnp.bfloat16).astype(jnp.float32)
                            contrib = t * w
                            if acc[co][g] is None:
                                acc[co][g] = contrib
                            else:
                                acc[co][g] = acc[co][g] + contrib

    for co in range(4):
        qs = [jnp.maximum(acc[co][g] + cb_ref[co], 0.0)
              .astype(jnp.bfloat16).astype(jnp.float32) for g in range(4)]
        pool_ref[pl.ds(co * 8, 8), :] = jnp.maximum(
            jnp.maximum(qs[0], qs[1]), jnp.maximum(qs[2], qs[3]))


def _head_body(lw_ref, f_ref, lb_ref, pose_ref, ang_ref, tick_ref,
               traj_ref, npose_ref):
    t = jnp.dot(lw_ref[...], f_ref[...],
                preferred_element_type=jnp.float32)
    t0 = t[0:1, :] + lb_ref[0]
    t1 = t[1:2, :] + lb_ref[1]

    q1 = jnp.full((1, 1), ang_ref[0], jnp.float32)
    q2 = jnp.full((1, 1), ang_ref[1], jnp.float32)
    ri = jax.lax.broadcasted_iota(jnp.int32, (2, 2), 0)
    ci = jax.lax.broadcasted_iota(jnp.int32, (2, 2), 1)
    rv = jax.lax.broadcasted_iota(jnp.int32, (2, 1), 0)
    for _ in range(_NUM_IK_STEPS):
        s1, c1 = jnp.sin(q1), jnp.cos(q1)
        q12 = q1 + q2
        s12, c12 = jnp.sin(q12), jnp.cos(q12)
        ex = t0 - (c1 + c12)
        ey = t1 - (s1 + s12)
        j11 = -s1 - s12
        j12 = -s12
        j21 = c1 + c12
        j22 = c12
        det = j11 * j22 - j12 * j21
        inv = 1.0 / (det + 1e-6)
        a11 = j22 * inv
        a12 = (-j12) * inv
        a21 = (-j21) * inv
        a22 = j11 * inv
        invj = jnp.where(ri == 0, jnp.where(ci == 0, a11, a12),
                         jnp.where(ci == 0, a21, a22))
        err = jnp.where(rv == 0, ex, ey)
        dq = jnp.dot(invj, err, preferred_element_type=jnp.float32)
        q1 = q1 + dq[0:1, :]
        q2 = q2 + dq[1:2, :]

    s1, c1 = jnp.sin(q1), jnp.cos(q1)
    q12 = q1 + q2
    s12, c12 = jnp.sin(q12), jnp.cos(q12)
    sx = c1 + c12
    sy = s1 + s12
    kf = jax.lax.broadcasted_iota(jnp.int32, (19, 2), 0).astype(jnp.float32)
    col = jax.lax.broadcasted_iota(jnp.int32, (19, 2), 1)
    t1v = jnp.minimum(kf, 9.0) * (1.0 / 9.0)
    t2v = jnp.maximum(kf - 9.0, 0.0) * (1.0 / 9.0)
    traj_ref[...] = jnp.where(col == 0, sx - 0.2 * t1v, sy - 0.1 * t2v)

    d_l = tick_ref[0] * 1e-4
    d_r = tick_ref[1] * 1e-4
    d_c = (d_l + d_r) * 0.5
    d_th = (d_r - d_l) * 2.0
    avg = jnp.full((1, 1), pose_ref[2] + d_th * 0.5, jnp.float32)
    npose_ref[0] = pose_ref[0] + d_c * jnp.cos(avg)[0, 0]
    npose_ref[1] = pose_ref[1] + d_c * jnp.sin(avg)[0, 0]
    npose_ref[2] = pose_ref[2] + d_th


def kernel(sensor_input, initial_pose, initial_angles, wheel_ticks,
           conv_w, conv_b, lin_w, lin_b):
    x = sensor_input.reshape(192, 64)
    vmem = pl.BlockSpec(memory_space=pltpu.VMEM)
    smem = pl.BlockSpec(memory_space=pltpu.SMEM)
    pool = pl.pallas_call(
        _conv_pool_body,
        in_specs=[vmem, smem, smem],
        out_specs=vmem,
        out_shape=jax.ShapeDtypeStruct((32, 8), jnp.float32),
    )(x, conv_w, conv_b)
    feat_col = pool.reshape(256, 1)
    trajectory, new_pose = pl.pallas_call(
        _head_body,
        in_specs=[vmem, vmem, smem, smem, smem, smem],
        out_specs=[vmem, smem],
        out_shape=[jax.ShapeDtypeStruct((19, 2), jnp.float32),
                   jax.ShapeDtypeStruct((3,), jnp.float32)],
    )(lin_w, feat_col, lin_b, initial_pose, initial_angles, wheel_ticks)
    return trajectory, new_pose


# speedup vs baseline: 4.9385x; 1.0133x over previous
import jax
import jax.numpy as jnp
from jax.experimental import pallas as pl
from jax.experimental.pallas import tpu as pltpu

_NUM_IK_STEPS = 9


def _conv_pool_body(x_ref, cw_ref, cb_ref, pool_ref):
    ri = jax.lax.broadcasted_iota(jnp.int32, (64, 72), 0)
    mi = jax.lax.broadcasted_iota(jnp.int32, (64, 72), 1)
    tgt = jnp.where(mi < 64, 8 * (mi % 8) + mi // 8, 8 * (mi - 64) - 1)
    sel = (ri == tgt).astype(jnp.float32)

    def colgroup(y, coff):
        if coff >= 0:
            return y[:, coff * 8:coff * 8 + 8]
        return y[:, 64:72]

    acc = [[None] * 4 for _ in range(4)]
    zrow = jnp.zeros((1, 64), jnp.float32)
    for ci in range(3):
        rows = [x_ref[pl.ds(ci * 64 + r, 8, 8), :]
                .astype(jnp.bfloat16).astype(jnp.float32) for r in range(8)]
        rows_m1 = jnp.concatenate([zrow, rows[7][:7, :]], axis=0)
        ys = {-1: jnp.dot(rows_m1, sel, preferred_element_type=jnp.float32)}
        for r in range(7):
            ys[r] = jnp.dot(rows[r], sel,
                            preferred_element_type=jnp.float32)
        for di in range(2):
            for dj in range(2):
                g = di * 2 + dj
                for dy in range(4):
                    for dx in range(4):
                        t = colgroup(ys[4 * di + dy - 1], 4 * dj + dx - 1)
                        for co in range(4):
                            w = cw_ref[co, ci, dy, dx].astype(
                                jnp.bfloat16).astype(jnp.float32)
                            contrib = t * w
                            if acc[co][g] is None:
                                acc[co][g] = contrib
                            else:
                                acc[co][g] = acc[co][g] + contrib

    for co in range(4):
        qs = [jnp.maximum(acc[co][g] + cb_ref[co], 0.0)
              .astype(jnp.bfloat16).astype(jnp.float32) for g in range(4)]
        pool_ref[pl.ds(co * 8, 8), :] = jnp.maximum(
            jnp.maximum(qs[0], qs[1]), jnp.maximum(qs[2], qs[3]))


def _head_body(lw_ref, f_ref, lb_ref, pose_ref, ang_ref, tick_ref,
               traj_ref, npose_ref):
    t = jnp.dot(lw_ref[...], f_ref[...],
                preferred_element_type=jnp.float32)
    t0 = t[0:1, :] + lb_ref[0]
    t1 = t[1:2, :] + lb_ref[1]

    q1 = jnp.full((1, 1), ang_ref[0], jnp.float32)
    q2 = jnp.full((1, 1), ang_ref[1], jnp.float32)
    ri = jax.lax.broadcasted_iota(jnp.int32, (2, 2), 0)
    ci = jax.lax.broadcasted_iota(jnp.int32, (2, 2), 1)
    rv = jax.lax.broadcasted_iota(jnp.int32, (2, 1), 0)
    for _ in range(_NUM_IK_STEPS):
        s1, c1 = jnp.sin(q1), jnp.cos(q1)
        q12 = q1 + q2
        s12, c12 = jnp.sin(q12), jnp.cos(q12)
        ex = t0 - (c1 + c12)
        ey = t1 - (s1 + s12)
        j11 = -s1 - s12
        j12 = -s12
        j21 = c1 + c12
        j22 = c12
        det = j11 * j22 - j12 * j21
        inv = 1.0 / (det + 1e-6)
        a11 = j22 * inv
        a12 = (-j12) * inv
        a21 = (-j21) * inv
        a22 = j11 * inv
        invj = jnp.where(ri == 0, jnp.where(ci == 0, a11, a12),
                         jnp.where(ci == 0, a21, a22))
        err = jnp.where(rv == 0, ex, ey)
        dq = jnp.dot(invj, err, preferred_element_type=jnp.float32)
        q1 = q1 + dq[0:1, :]
        q2 = q2 + dq[1:2, :]

    s1, c1 = jnp.sin(q1), jnp.cos(q1)
    q12 = q1 + q2
    s12, c12 = jnp.sin(q12), jnp.cos(q12)
    sx = c1 + c12
    sy = s1 + s12
    kf = jax.lax.broadcasted_iota(jnp.int32, (19, 2), 0).astype(jnp.float32)
    col = jax.lax.broadcasted_iota(jnp.int32, (19, 2), 1)
    t1v = jnp.minimum(kf, 9.0) * (1.0 / 9.0)
    t2v = jnp.maximum(kf - 9.0, 0.0) * (1.0 / 9.0)
    traj_ref[...] = jnp.where(col == 0, sx - 0.2 * t1v, sy - 0.1 * t2v)

    d_l = tick_ref[0] * 1e-4
    d_r = tick_ref[1] * 1e-4
    d_c = (d_l + d_r) * 0.5
    d_th = (d_r - d_l) * 2.0
    avg = jnp.full((1, 1), pose_ref[2] + d_th * 0.5, jnp.float32)
    npose_ref[0] = pose_ref[0] + d_c * jnp.cos(avg)[0, 0]
    npose_ref[1] = pose_ref[1] + d_c * jnp.sin(avg)[0, 0]
    npose_ref[2] = pose_ref[2] + d_th


def kernel(sensor_input, initial_pose, initial_angles, wheel_ticks,
           conv_w, conv_b, lin_w, lin_b):
    x = sensor_input.reshape(192, 64)
    vmem = pl.BlockSpec(memory_space=pltpu.VMEM)
    smem = pl.BlockSpec(memory_space=pltpu.SMEM)
    pool = pl.pallas_call(
        _conv_pool_body,
        in_specs=[vmem, smem, smem],
        out_specs=vmem,
        out_shape=jax.ShapeDtypeStruct((32, 8), jnp.float32),
    )(x, conv_w, conv_b)
    feat_col = pool.reshape(256, 1)
    trajectory, new_pose = pl.pallas_call(
        _head_body,
        in_specs=[vmem, vmem, smem, smem, smem, smem],
        out_specs=[vmem, smem],
        out_shape=[jax.ShapeDtypeStruct((19, 2), jnp.float32),
                   jax.ShapeDtypeStruct((3,), jnp.float32)],
    )(lin_w, feat_col, lin_b, initial_pose, initial_angles, wheel_ticks)
    return trajectory, new_pose
